# baseline (speedup 1.0000x reference)
import functools

import jax
import jax.numpy as jnp
import numpy as np
from jax import lax
from jax.experimental import pallas as pl
from jax.experimental.pallas import tpu as pltpu
from jax.experimental.pallas import tpu_sc as plsc

NC = 2
NS = 16
NW = NC * NS
CHUNK = 128
LW = 128

GS_MF = 1 << 15
GS_ML = 1 << 17


def _repack2(tTu, tTi, gs):
    d, n = tTu.shape
    ng = 2 * LW // d
    rows = 4096
    cols = rows
    grid = gs // rows
    nblk = -(-n // cols)

    def one(xs):
        x = jnp.concatenate([x[...] for x in xs], axis=0)
        y = x.astype(jnp.bfloat16).T
        return pltpu.bitcast(y.reshape(2 * cols, LW), jnp.float32)

    def body(*refs):
        xs, ou, oi = refs[:-2], refs[-2], refs[-1]
        ou[...] = one(xs[:ng])
        oi[...] = one(xs[ng:])

    spec = lambda q: pl.BlockSpec(
        (d, cols),
        functools.partial(
            lambda i, q=q: (0, jnp.minimum(q * (gs // cols) + i, nblk - 1))))
    in_specs = [spec(q) for q in range(ng)] * 2
    out_spec = pl.BlockSpec((rows, LW), lambda i: (i, 0))
    out_sds = jax.ShapeDtypeStruct((gs, LW), jnp.float32)
    return pl.pallas_call(
        body,
        grid=(grid,),
        in_specs=in_specs,
        out_specs=[out_spec, out_spec],
        out_shape=[out_sds, out_sds],
    )(*([tTu] * ng), *([tTi] * ng))


def _sc_gather2(user, item, tbl_u, tbl_i, gs):
    B = user.shape[0]
    bpw = B // NW
    nch = bpw // CHUNK
    mesh = plsc.VectorSubcoreMesh(core_axis_name="c", subcore_axis_name="s")
    out_sds = jax.ShapeDtypeStruct((B, LW), jnp.float32)

    @functools.partial(
        pl.kernel,
        mesh=mesh,
        compiler_params=pltpu.CompilerParams(use_tc_tiling_on_sc=True),
        out_type=[out_sds] * 2,
        scratch_types=[
            pltpu.VMEM((nch, CHUNK), jnp.int32),
            pltpu.VMEM((nch, CHUNK), jnp.int32),
            pltpu.VMEM((nch, CHUNK, LW), jnp.float32),
            pltpu.SemaphoreType.DMA,
            pltpu.SemaphoreType.DMA,
        ],
    )
    def k(user_hbm, item_hbm, h_u, h_i, o_u, o_i,
          ridx_u, ridx_i, ring, sem_g, sem_o):
        wid = lax.axis_index("s") * NC + lax.axis_index("c")
        base = wid * bpw
        for j in range(nch):
            pltpu.sync_copy(user_hbm.at[pl.ds(base + j * CHUNK, CHUNK)], ridx_u.at[j])
            pltpu.sync_copy(item_hbm.at[pl.ds(base + j * CHUNK, CHUNK)], ridx_i.at[j])
        for j in range(nch):
            for o in range(0, CHUNK, 16):
                s = pl.ds(o, 16)
                ridx_u[j, s] = ridx_u[j, s] & (gs - 1)
                ridx_i[j, s] = ridx_i[j, s] & (gs - 1)
        work = [(h_u, ridx_u, o_u, c) for c in range(nch)]
        work += [(h_i, ridx_i, o_i, c) for c in range(nch)]

        def gather(w):
            tbl, ridx, _, c = w
            return pltpu.async_copy(tbl.at[ridx.at[c]], ring.at[c % nch], sem_g)

        def outcopy(w):
            _, _, out, c = w
            return pltpu.async_copy(ring.at[c % nch],
                                    out.at[pl.ds(base + c * CHUNK, CHUNK)], sem_o)

        gs_ = [gather(w) for w in work[:nch]]
        os_ = []
        for k_ in range(len(work)):
            gs_[k_].wait()
            os_.append(outcopy(work[k_]))
            nxt = k_ + nch
            if nxt < len(work):
                os_[k_].wait()
                gs_.append(gather(work[nxt]))
        for k_ in range(len(work) - nch, len(work)):
            os_[k_].wait()

    return k(user, item, tbl_u, tbl_i)


def _unpack(ref):
    x = pltpu.bitcast(ref[...], jnp.bfloat16)
    return x.reshape(x.shape[0] // 2, 2 * LW).astype(jnp.float32)


def _tc_body(u_ref, it_ref, mfu_ref, mfi_ref, mlu_ref, mli_ref,
             w0u_ref, w0i_ref, b0_ref, w1_ref, b1_ref, w2_ref, b2_ref,
             s8_ref, wpm_ref, wph_ref, bp_ref, out_ref):
    f32 = jnp.float32
    R = mfu_ref.shape[0]
    col = lax.broadcasted_iota(jnp.int32, (R, 2 * LW), 1)
    u = u_ref[...]
    it = it_ref[...]
    zero = jnp.zeros((), f32)
    xu = jnp.where((col >> 5) == (u >> 17), _unpack(mlu_ref), zero)
    xi = jnp.where((col >> 5) == (it >> 17), _unpack(mli_ref), zero)
    h = jnp.dot(xu, w0u_ref[...], preferred_element_type=f32)
    h = h + jnp.dot(xi, w0i_ref[...], preferred_element_type=f32)
    h = jnp.maximum(h + b0_ref[...], 0.0)
    h = jnp.maximum(jnp.dot(h, w1_ref[...], preferred_element_type=f32) + b1_ref[...], 0.0)
    h = jnp.maximum(jnp.dot(h, w2_ref[...], preferred_element_type=f32) + b2_ref[...], 0.0)
    gu = jnp.where((col >> 3) == (u >> 15), _unpack(mfu_ref), zero)
    gi = jnp.where((col >> 3) == (it >> 15), _unpack(mfi_ref), zero)
    mfu_x = jnp.dot(gu, s8_ref[...], preferred_element_type=f32)
    mfi_x = jnp.dot(gi, s8_ref[...], preferred_element_type=f32)
    mf = mfu_x * mfi_x
    logit = (jnp.dot(mf, wpm_ref[...], preferred_element_type=f32)
             + jnp.dot(h, wph_ref[...], preferred_element_type=f32)
             + bp_ref[...])
    out_ref[...] = jax.nn.sigmoid(logit) * 4.5 + 0.5


def kernel(user, item, mf_user_emb, mf_item_emb, mlp_user_emb, mlp_item_emb,
           w0, b0, w1, b1, w2, b2, wp, bp):
    B = user.shape[0]
    dmf = mf_user_emb.shape[1]
    dml = mlp_user_emb.shape[1]

    p_mlu, p_mli = _repack2(mlp_user_emb.T, mlp_item_emb.T, GS_ML)
    mlu, mli = _sc_gather2(user, item, p_mlu, p_mli, GS_ML)
    p_mfu, p_mfi = _repack2(mf_user_emb.T, mf_item_emb.T, GS_MF)
    H = B // 2
    mf_halves = [_sc_gather2(user[h * H:(h + 1) * H], item[h * H:(h + 1) * H],
                             p_mfu, p_mfi, GS_MF) for h in range(2)]

    w0u = jnp.tile(w0[:, :dml].T, (2 * LW // dml, 1))
    w0i = jnp.tile(w0[:, dml:].T, (2 * LW // dml, 1))
    s8 = jnp.asarray(np.tile(np.eye(dmf, dtype=np.float32),
                             (2 * LW // dmf, 1)))
    w1t = w1.T
    w2t = w2.T
    wpm = wp[:, :dmf].T
    wph = wp[:, dmf:].T
    b0r = b0.reshape(1, -1)
    b1r = b1.reshape(1, -1)
    b2r = b2.reshape(1, -1)
    bpr = bp.reshape(1, 1)
    u2d = user.reshape(B, 1)
    i2d = item.reshape(B, 1)

    R = 2048
    d0 = w0.shape[0]
    d1 = w1.shape[0]
    d2 = w2.shape[0]
    data = lambda c: pl.BlockSpec((R, c), lambda i: (i, 0))
    full = lambda a, b: pl.BlockSpec((a, b), lambda i: (0, 0))
    halves = []
    for h in range(2):
        s = slice(h * H, (h + 1) * H)
        mfu, mfi = mf_halves[h]
        out2 = pl.pallas_call(
            _tc_body,
            grid=(H // R,),
            in_specs=[
                data(1), data(1), data(LW), data(LW), data(LW), data(LW),
                full(2 * LW, d0), full(2 * LW, d0), full(1, d0),
                full(d0, d1), full(1, d1),
                full(d1, d2), full(1, d2),
                full(2 * LW, dmf),
                full(dmf, 1), full(d2, 1), full(1, 1),
            ],
            out_specs=pl.BlockSpec((R, 1), lambda i: (i, 0)),
            out_shape=jax.ShapeDtypeStruct((H, 1), jnp.float32),
        )(u2d[s], i2d[s], mfu, mfi, mlu[s], mli[s], w0u, w0i, b0r, w1t,
          b1r, w2t, b2r, s8, wpm, wph, bpr)
        halves.append(out2.reshape(H))
    return jnp.concatenate(halves)


# speedup vs baseline: 1.0918x; 1.0918x over previous
import functools

import jax
import jax.numpy as jnp
import numpy as np
from jax import lax
from jax.experimental import pallas as pl
from jax.experimental.pallas import tpu as pltpu
from jax.experimental.pallas import tpu_sc as plsc

NC = 2
NS = 16
NW = NC * NS
CHUNK = 128
LW = 128

GS_MF = 1 << 15
GS_ML = 1 << 17


def _repack2(tTu, tTi, gs):
    d, n = tTu.shape
    ng = 2 * LW // d
    rows = 4096
    cols = rows
    grid = gs // rows
    nblk = -(-n // cols)

    def one(xs):
        x = jnp.concatenate([x[...] for x in xs], axis=0)
        y = x.astype(jnp.bfloat16).T
        return pltpu.bitcast(y.reshape(2 * cols, LW), jnp.float32)

    def body(*refs):
        xs, ou, oi = refs[:-2], refs[-2], refs[-1]
        ou[...] = one(xs[:ng])
        oi[...] = one(xs[ng:])

    spec = lambda q: pl.BlockSpec(
        (d, cols),
        functools.partial(
            lambda i, q=q: (0, jnp.minimum(q * (gs // cols) + i, nblk - 1))))
    in_specs = [spec(q) for q in range(ng)] * 2
    out_spec = pl.BlockSpec((rows, LW), lambda i: (i, 0))
    out_sds = jax.ShapeDtypeStruct((gs, LW), jnp.float32)
    return pl.pallas_call(
        body,
        grid=(grid,),
        in_specs=in_specs,
        out_specs=[out_spec, out_spec],
        out_shape=[out_sds, out_sds],
    )(*([tTu] * ng), *([tTi] * ng))


def _sc_gather2(user, item, tbl_u, tbl_i, gs):
    B = user.shape[0]
    bpw = B // NW
    nch = bpw // CHUNK
    mesh = plsc.VectorSubcoreMesh(core_axis_name="c", subcore_axis_name="s")
    out_sds = jax.ShapeDtypeStruct((B, LW), jnp.float32)

    @functools.partial(
        pl.kernel,
        mesh=mesh,
        compiler_params=pltpu.CompilerParams(use_tc_tiling_on_sc=True),
        out_type=[out_sds] * 2,
        scratch_types=[
            pltpu.VMEM((nch, CHUNK), jnp.int32),
            pltpu.VMEM((nch, CHUNK), jnp.int32),
            pltpu.VMEM((nch, CHUNK, LW), jnp.float32),
            pltpu.SemaphoreType.DMA,
            pltpu.SemaphoreType.DMA,
        ],
    )
    def k(user_hbm, item_hbm, h_u, h_i, o_u, o_i,
          ridx_u, ridx_i, ring, sem_g, sem_o):
        wid = lax.axis_index("s") * NC + lax.axis_index("c")
        base = wid * bpw
        for j in range(nch):
            pltpu.sync_copy(user_hbm.at[pl.ds(base + j * CHUNK, CHUNK)], ridx_u.at[j])
            pltpu.sync_copy(item_hbm.at[pl.ds(base + j * CHUNK, CHUNK)], ridx_i.at[j])
        for j in range(nch):
            for o in range(0, CHUNK, 16):
                s = pl.ds(o, 16)
                ridx_u[j, s] = ridx_u[j, s] & (gs - 1)
                ridx_i[j, s] = ridx_i[j, s] & (gs - 1)
        work = [(h_u, ridx_u, o_u, c) for c in range(nch)]
        work += [(h_i, ridx_i, o_i, c) for c in range(nch)]

        def gather(w):
            tbl, ridx, _, c = w
            return pltpu.async_copy(tbl.at[ridx.at[c]], ring.at[c % nch], sem_g)

        def outcopy(w):
            _, _, out, c = w
            return pltpu.async_copy(ring.at[c % nch],
                                    out.at[pl.ds(base + c * CHUNK, CHUNK)], sem_o)

        gs_ = [gather(w) for w in work[:nch]]
        os_ = []
        for k_ in range(len(work)):
            gs_[k_].wait()
            os_.append(outcopy(work[k_]))
            nxt = k_ + nch
            if nxt < len(work):
                os_[k_].wait()
                gs_.append(gather(work[nxt]))
        for k_ in range(len(work) - nch, len(work)):
            os_[k_].wait()

    return k(user, item, tbl_u, tbl_i)


def _unpack(ref):
    x = pltpu.bitcast(ref[...], jnp.bfloat16)
    return x.reshape(x.shape[0] // 2, 2 * LW).astype(jnp.float32)


def _tc_body(u_ref, it_ref, mfu_ref, mfi_ref, mlu_ref, mli_ref,
             w0u_ref, w0i_ref, b0_ref, w1_ref, b1_ref, w2_ref, b2_ref,
             s8_ref, wpm_ref, wph_ref, bp_ref, out_ref):
    f32 = jnp.float32
    R = mfu_ref.shape[0]
    col = lax.broadcasted_iota(jnp.int32, (R, 2 * LW), 1)
    u = u_ref[...]
    it = it_ref[...]
    zero = jnp.zeros((), f32)
    xu = jnp.where((col >> 5) == (u >> 17), _unpack(mlu_ref), zero)
    xi = jnp.where((col >> 5) == (it >> 17), _unpack(mli_ref), zero)
    h = jnp.dot(xu, w0u_ref[...], preferred_element_type=f32)
    h = h + jnp.dot(xi, w0i_ref[...], preferred_element_type=f32)
    h = jnp.maximum(h + b0_ref[...], 0.0)
    h = jnp.maximum(jnp.dot(h, w1_ref[...], preferred_element_type=f32) + b1_ref[...], 0.0)
    h = jnp.maximum(jnp.dot(h, w2_ref[...], preferred_element_type=f32) + b2_ref[...], 0.0)
    gu = jnp.where((col >> 3) == (u >> 15), _unpack(mfu_ref), zero)
    gi = jnp.where((col >> 3) == (it >> 15), _unpack(mfi_ref), zero)
    mfu_x = jnp.dot(gu, s8_ref[...], preferred_element_type=f32)
    mfi_x = jnp.dot(gi, s8_ref[...], preferred_element_type=f32)
    mf = mfu_x * mfi_x
    logit = (jnp.dot(mf, wpm_ref[...], preferred_element_type=f32)
             + jnp.dot(h, wph_ref[...], preferred_element_type=f32)
             + bp_ref[...])
    out_ref[...] = jax.nn.sigmoid(logit) * 4.5 + 0.5


def kernel(user, item, mf_user_emb, mf_item_emb, mlp_user_emb, mlp_item_emb,
           w0, b0, w1, b1, w2, b2, wp, bp):
    B = user.shape[0]
    dmf = mf_user_emb.shape[1]
    dml = mlp_user_emb.shape[1]

    p_mlu, p_mli = _repack2(mlp_user_emb.T, mlp_item_emb.T, GS_ML)
    mlu, mli = _sc_gather2(user, item, p_mlu, p_mli, GS_ML)
    p_mfu, p_mfi = _repack2(mf_user_emb.T, mf_item_emb.T, GS_MF)
    mfu, mfi = _sc_gather2(user, item, p_mfu, p_mfi, GS_MF)

    w0u = jnp.tile(w0[:, :dml].T, (2 * LW // dml, 1))
    w0i = jnp.tile(w0[:, dml:].T, (2 * LW // dml, 1))
    s8 = jnp.asarray(np.tile(np.eye(dmf, dtype=np.float32),
                             (2 * LW // dmf, 1)))
    w1t = w1.T
    w2t = w2.T
    wpm = wp[:, :dmf].T
    wph = wp[:, dmf:].T
    b0r = b0.reshape(1, -1)
    b1r = b1.reshape(1, -1)
    b2r = b2.reshape(1, -1)
    bpr = bp.reshape(1, 1)
    u2d = user.reshape(B, 1)
    i2d = item.reshape(B, 1)

    R = 2048
    d0 = w0.shape[0]
    d1 = w1.shape[0]
    d2 = w2.shape[0]
    data = lambda c: pl.BlockSpec((R, c), lambda i: (i, 0))
    full = lambda a, b: pl.BlockSpec((a, b), lambda i: (0, 0))
    out2 = pl.pallas_call(
        _tc_body,
        grid=(B // R,),
        in_specs=[
            data(1), data(1), data(LW), data(LW), data(LW), data(LW),
            full(2 * LW, d0), full(2 * LW, d0), full(1, d0),
            full(d0, d1), full(1, d1),
            full(d1, d2), full(1, d2),
            full(2 * LW, dmf),
            full(dmf, 1), full(d2, 1), full(1, 1),
        ],
        out_specs=pl.BlockSpec((R, 1), lambda i: (i, 0)),
        out_shape=jax.ShapeDtypeStruct((B, 1), jnp.float32),
    )(u2d, i2d, mfu, mfi, mlu, mli, w0u, w0i, b0r, w1t, b1r, w2t, b2r,
      s8, wpm, wph, bpr)
    return out2.reshape(B)


# speedup vs baseline: 1.0935x; 1.0015x over previous
import functools

import jax
import jax.numpy as jnp
import numpy as np
from jax import lax
from jax.experimental import pallas as pl
from jax.experimental.pallas import tpu as pltpu
from jax.experimental.pallas import tpu_sc as plsc

NC = 2
NS = 16
NW = NC * NS
CHUNK = 128
LW = 128

GS_MF = 1 << 15
GS_ML = 1 << 17


def _repack2(tTu, tTi, gs):
    d, n = tTu.shape
    ng = 2 * LW // d
    rows = 4096
    cols = rows
    grid = gs // rows
    nblk = -(-n // cols)

    def one(xs):
        x = jnp.concatenate([x[...] for x in xs], axis=0)
        y = x.astype(jnp.bfloat16).T
        return pltpu.bitcast(y.reshape(2 * cols, LW), jnp.float32)

    def body(*refs):
        xs, ou, oi = refs[:-2], refs[-2], refs[-1]
        ou[...] = one(xs[:ng])
        oi[...] = one(xs[ng:])

    spec = lambda q: pl.BlockSpec(
        (d, cols),
        functools.partial(
            lambda i, q=q: (0, jnp.minimum(q * (gs // cols) + i, nblk - 1))))
    in_specs = [spec(q) for q in range(ng)] * 2
    out_spec = pl.BlockSpec((rows, LW), lambda i: (i, 0))
    out_sds = jax.ShapeDtypeStruct((gs, LW), jnp.float32)
    return pl.pallas_call(
        body,
        grid=(grid,),
        in_specs=in_specs,
        out_specs=[out_spec, out_spec],
        out_shape=[out_sds, out_sds],
    )(*([tTu] * ng), *([tTi] * ng))


def _sc_gather2(user, item, tbl_u, tbl_i, gs):
    B = user.shape[0]
    bpw = B // NW
    nch = bpw // CHUNK
    mesh = plsc.VectorSubcoreMesh(core_axis_name="c", subcore_axis_name="s")
    out_sds = jax.ShapeDtypeStruct((B, LW), jnp.float32)

    @functools.partial(
        pl.kernel,
        mesh=mesh,
        compiler_params=pltpu.CompilerParams(use_tc_tiling_on_sc=True),
        out_type=[out_sds] * 2,
        scratch_types=[
            pltpu.VMEM((nch, CHUNK), jnp.int32),
            pltpu.VMEM((nch, CHUNK), jnp.int32),
            pltpu.VMEM((nch, CHUNK, LW), jnp.float32),
            pltpu.SemaphoreType.DMA,
            pltpu.SemaphoreType.DMA,
        ],
    )
    def k(user_hbm, item_hbm, h_u, h_i, o_u, o_i,
          ridx_u, ridx_i, ring, sem_g, sem_o):
        wid = lax.axis_index("s") * NC + lax.axis_index("c")
        base = wid * bpw
        for j in range(nch):
            pltpu.sync_copy(user_hbm.at[pl.ds(base + j * CHUNK, CHUNK)], ridx_u.at[j])
            pltpu.sync_copy(item_hbm.at[pl.ds(base + j * CHUNK, CHUNK)], ridx_i.at[j])
        for j in range(nch):
            for o in range(0, CHUNK, 16):
                s = pl.ds(o, 16)
                ridx_u[j, s] = ridx_u[j, s] & (gs - 1)
                ridx_i[j, s] = ridx_i[j, s] & (gs - 1)
        work = [(h_u, ridx_u, o_u, c) for c in range(nch)]
        work += [(h_i, ridx_i, o_i, c) for c in range(nch)]

        def gather(w):
            tbl, ridx, _, c = w
            return pltpu.async_copy(tbl.at[ridx.at[c]], ring.at[c % nch], sem_g)

        def outcopy(w):
            _, _, out, c = w
            return pltpu.async_copy(ring.at[c % nch],
                                    out.at[pl.ds(base + c * CHUNK, CHUNK)], sem_o)

        gs_ = [gather(w) for w in work[:nch]]
        os_ = []
        for k_ in range(len(work)):
            gs_[k_].wait()
            os_.append(outcopy(work[k_]))
            nxt = k_ + nch
            if nxt < len(work):
                os_[k_].wait()
                gs_.append(gather(work[nxt]))
        for k_ in range(len(work) - nch, len(work)):
            os_[k_].wait()

    return k(user, item, tbl_u, tbl_i)


def _unpack(ref):
    x = pltpu.bitcast(ref[...], jnp.bfloat16)
    return x.reshape(x.shape[0] // 2, 2 * LW).astype(jnp.float32)


def _tc_body(u_ref, it_ref, mfu_ref, mfi_ref, mlu_ref, mli_ref,
             w0u_ref, w0i_ref, b0_ref, w1_ref, b1_ref, w2_ref, b2_ref,
             s8_ref, wpm_ref, wph_ref, bp_ref, out_ref):
    f32 = jnp.float32
    R = mfu_ref.shape[0]
    col = lax.broadcasted_iota(jnp.int32, (R, 2 * LW), 1)
    u = u_ref[...]
    it = it_ref[...]
    zero = jnp.zeros((), f32)
    xu = jnp.where((col >> 5) == (u >> 17), _unpack(mlu_ref), zero)
    xi = jnp.where((col >> 5) == (it >> 17), _unpack(mli_ref), zero)
    h = jnp.dot(xu, w0u_ref[...], preferred_element_type=f32)
    h = h + jnp.dot(xi, w0i_ref[...], preferred_element_type=f32)
    h = jnp.maximum(h + b0_ref[...], 0.0)
    h = jnp.maximum(jnp.dot(h, w1_ref[...], preferred_element_type=f32) + b1_ref[...], 0.0)
    h = jnp.maximum(jnp.dot(h, w2_ref[...], preferred_element_type=f32) + b2_ref[...], 0.0)
    gu = jnp.where((col >> 3) == (u >> 15), _unpack(mfu_ref), zero)
    gi = jnp.where((col >> 3) == (it >> 15), _unpack(mfi_ref), zero)
    mfu_x = jnp.dot(gu, s8_ref[...], preferred_element_type=f32)
    mfi_x = jnp.dot(gi, s8_ref[...], preferred_element_type=f32)
    mf = mfu_x * mfi_x
    logit = (jnp.dot(mf, wpm_ref[...], preferred_element_type=f32)
             + jnp.dot(h, wph_ref[...], preferred_element_type=f32)
             + bp_ref[...])
    out_ref[...] = jax.nn.sigmoid(logit) * 4.5 + 0.5


def kernel(user, item, mf_user_emb, mf_item_emb, mlp_user_emb, mlp_item_emb,
           w0, b0, w1, b1, w2, b2, wp, bp):
    B = user.shape[0]
    dmf = mf_user_emb.shape[1]
    dml = mlp_user_emb.shape[1]

    p_mlu, p_mli = _repack2(mlp_user_emb.T, mlp_item_emb.T, GS_ML)
    mlu, mli = _sc_gather2(user, item, p_mlu, p_mli, GS_ML)
    p_mfu, p_mfi = _repack2(mf_user_emb.T, mf_item_emb.T, GS_MF)
    mfu, mfi = _sc_gather2(user, item, p_mfu, p_mfi, GS_MF)

    w0u = jnp.tile(w0[:, :dml].T, (2 * LW // dml, 1))
    w0i = jnp.tile(w0[:, dml:].T, (2 * LW // dml, 1))
    s8 = jnp.asarray(np.tile(np.eye(dmf, dtype=np.float32),
                             (2 * LW // dmf, 1)))
    w1t = w1.T
    w2t = w2.T
    wpm = wp[:, :dmf].T
    wph = wp[:, dmf:].T
    b0r = b0.reshape(1, -1)
    b1r = b1.reshape(1, -1)
    b2r = b2.reshape(1, -1)
    bpr = bp.reshape(1, 1)
    u2d = user.reshape(B, 1)
    i2d = item.reshape(B, 1)

    R = 4096
    d0 = w0.shape[0]
    d1 = w1.shape[0]
    d2 = w2.shape[0]
    data = lambda c: pl.BlockSpec((R, c), lambda i: (i, 0))
    full = lambda a, b: pl.BlockSpec((a, b), lambda i: (0, 0))
    out2 = pl.pallas_call(
        _tc_body,
        grid=(B // R,),
        in_specs=[
            data(1), data(1), data(LW), data(LW), data(LW), data(LW),
            full(2 * LW, d0), full(2 * LW, d0), full(1, d0),
            full(d0, d1), full(1, d1),
            full(d1, d2), full(1, d2),
            full(2 * LW, dmf),
            full(dmf, 1), full(d2, 1), full(1, 1),
        ],
        out_specs=pl.BlockSpec((R, 1), lambda i: (i, 0)),
        out_shape=jax.ShapeDtypeStruct((B, 1), jnp.float32),
    )(u2d, i2d, mfu, mfi, mlu, mli, w0u, w0i, b0r, w1t, b1r, w2t, b2r,
      s8, wpm, wph, bpr)
    return out2.reshape(B)
